# baseline (speedup 1.0000x reference)
import functools

import jax
import jax.numpy as jnp
from jax import lax
from jax.experimental import pallas as pl
from jax.experimental.pallas import tpu as pltpu
from jax.experimental.pallas import tpu_sc as plsc

BETA = 0.25


def _argmin_body(x_ref, e_ref, idx_ref, loss_ref, esq_ref, em2_ref, w_ref,
                 *, RB, KB, NK, K, scale):
    i = pl.program_id(0)

    @pl.when(i == 0)
    def _():
        e = e_ref[...]
        esq_ref[...] = jnp.sum(e * e, axis=0, keepdims=True)
        em2_ref[...] = (e * -2.0).astype(jnp.bfloat16)
        loss_ref[...] = jnp.zeros_like(loss_ref)
        kk = lax.broadcasted_iota(jnp.int32, (KB, 128), 0)
        cc = lax.broadcasted_iota(jnp.int32, (KB, 128), 1)
        w = jnp.where(cc == 0, 1,
                      jnp.where(cc == 1, kk >> 6,
                                jnp.where(cc == 2, kk & 63, 0)))
        w_ref[...] = w.astype(jnp.bfloat16)

    x = x_ref[...]
    xb = x.astype(jnp.bfloat16)
    xsq = jnp.sum(x * x, axis=1, keepdims=True)

    def dist_at(kb):
        off = kb * KB
        sim2 = lax.dot_general(
            xb, em2_ref[:, pl.ds(off, KB)], (((1,), (0,)), ((), ())),
            preferred_element_type=jnp.float32)
        return (xsq + esq_ref[:, pl.ds(off, KB)]) + sim2

    def fast_step(kb, carry):
        rmin, rs, roff = carry
        d = dist_at(kb)
        m = jnp.min(d, axis=1, keepdims=True)
        mask = (d == m).astype(jnp.bfloat16)
        s = lax.dot_general(
            mask, w_ref[...], (((1,), (0,)), ((), ())),
            preferred_element_type=jnp.float32)
        better = m < rmin
        return (jnp.where(better, m, rmin),
                jnp.where(better, s, rs),
                jnp.where(better, jnp.full_like(roff, kb * KB), roff))

    rmin0 = jnp.full((RB, 1), jnp.inf, jnp.float32)
    rs0 = jnp.zeros((RB, 128), jnp.float32)
    roff0 = jnp.zeros((RB, 1), jnp.int32)
    rmin, rs, roff = lax.fori_loop(0, NK, fast_step, (rmin0, rs0, roff0))
    cnt = rs[:, 0:1]
    no_ties = jnp.all(cnt == 1.0)

    @pl.when(no_ties)
    def _():
        idx = (rs[:, 1:2] * 64.0 + rs[:, 2:3]).astype(jnp.int32) + roff
        idx_ref[...] = idx

    @pl.when(jnp.logical_not(no_ties))
    def _():
        def exact_step(kb, carry):
            emin, eidx = carry
            off = kb * KB
            d = dist_at(kb)
            m = jnp.min(d, axis=1, keepdims=True)
            iota = lax.broadcasted_iota(jnp.int32, (RB, KB), 1) + off
            bidx = jnp.min(jnp.where(d == m, iota, K), axis=1, keepdims=True)
            better = m < emin
            return jnp.where(better, m, emin), jnp.where(better, bidx, eidx)

        _, eidx = lax.fori_loop(
            0, NK, exact_step,
            (jnp.full((RB, 1), jnp.inf, jnp.float32),
             jnp.zeros((RB, 1), jnp.int32)))
        idx_ref[...] = eidx

    loss_ref[...] += jnp.sum(rmin, axis=0, keepdims=True) * scale


def _encode_indices(x, embedding, scale):
    N, D = x.shape
    K = embedding.shape[1]
    RB = 1024 if N % 1024 == 0 else N
    KB = 4096 if K % 4096 == 0 else K
    NR, NK = N // RB, K // KB
    body = functools.partial(_argmin_body, RB=RB, KB=KB, NK=NK, K=K, scale=scale)
    idx, loss = pl.pallas_call(
        body,
        grid=(NR,),
        in_specs=[
            pl.BlockSpec((RB, D), lambda i: (i, 0)),
            pl.BlockSpec((D, K), lambda i: (0, 0)),
        ],
        out_specs=[
            pl.BlockSpec((RB, 1), lambda i: (i, 0)),
            pl.BlockSpec((1, 1), lambda i: (0, 0)),
        ],
        out_shape=[
            jax.ShapeDtypeStruct((N, 1), jnp.int32),
            jax.ShapeDtypeStruct((1, 1), jnp.float32),
        ],
        scratch_shapes=[
            pltpu.VMEM((1, K), jnp.float32),
            pltpu.VMEM((D, K), jnp.bfloat16),
            pltpu.VMEM((KB, 128), jnp.bfloat16),
        ],
        compiler_params=pltpu.CompilerParams(
            dimension_semantics=("arbitrary",)),
    )(x, embedding)
    return idx.reshape(N), loss.reshape(())


def _sc_gather(e_t, idx, N, D):
    info = plsc.get_sparse_core_info()
    NC, NS = info.num_cores, info.num_subcores
    NW = NC * NS
    BPW = N // NW
    CH = min(BPW, 128)
    NCH = BPW // CH
    mesh = plsc.VectorSubcoreMesh(core_axis_name="c", subcore_axis_name="s")

    @functools.partial(
        pl.kernel,
        out_type=jax.ShapeDtypeStruct((N, D), jnp.float32),
        mesh=mesh,
        scratch_types=[
            [pltpu.VMEM((CH,), jnp.int32)] * NCH,
            [pltpu.VMEM((CH, D), jnp.float32)] * NCH,
            [pltpu.SemaphoreType.DMA] * NCH,
            [pltpu.SemaphoreType.DMA] * NCH,
        ],
    )
    def sc_kernel(et_hbm, idx_hbm, out_hbm, idx_vs, q_vs, gsems, wsems):
        wid = lax.axis_index("s") * NC + lax.axis_index("c")
        base = wid * BPW
        gathers = []
        for ch in range(NCH):
            cb = base + ch * CH
            pltpu.sync_copy(idx_hbm.at[pl.ds(cb, CH)], idx_vs[ch])
            gathers.append(pltpu.async_copy(
                et_hbm.at[idx_vs[ch]], q_vs[ch], gsems[ch]))
        writes = []
        for ch in range(NCH):
            cb = base + ch * CH
            gathers[ch].wait()
            writes.append(pltpu.async_copy(
                q_vs[ch], out_hbm.at[pl.ds(cb, CH)], wsems[ch]))
        for w in writes:
            w.wait()

    return sc_kernel(e_t, idx)


def kernel(inputs, embedding):
    orig_shape = inputs.shape
    x = inputs.reshape(-1, orig_shape[-1])
    N, D = x.shape
    e_t = embedding.T.astype(jnp.bfloat16).astype(jnp.float32)
    scale = (1.0 + BETA) / float(inputs.size)
    idx, loss = _encode_indices(x, embedding, scale)
    out = _sc_gather(e_t, idx, N, D)
    return out.reshape(orig_shape), loss


# speedup vs baseline: 1.3766x; 1.3766x over previous
import functools

import jax
import jax.numpy as jnp
from jax import lax
from jax.experimental import pallas as pl
from jax.experimental.pallas import tpu as pltpu
from jax.experimental.pallas import tpu_sc as plsc

BETA = 0.25


def _prep_body(e_ref, esq_ref, em2_ref):
    e = e_ref[...]
    esq_ref[...] = jnp.sum(e * e, axis=0, keepdims=True)
    em2_ref[...] = (e * -2.0).astype(jnp.bfloat16)


def _argmin_body(x_ref, esq_full_ref, em2_full_ref, idx_ref, loss_ref,
                 *, RB, KB, NK, K, scale):
    esq_ref = esq_full_ref
    em2_ref = em2_full_ref
    x = x_ref[...]
    xb = x.astype(jnp.bfloat16)
    xsq = jnp.sum(x * x, axis=1, keepdims=True)

    def dist_step(kb, carry):
        rmin, ridx = carry
        off = kb * KB
        sim2 = lax.dot_general(
            xb, em2_ref[:, pl.ds(off, KB)], (((1,), (0,)), ((), ())),
            preferred_element_type=jnp.float32)
        d = (xsq + esq_ref[:, pl.ds(off, KB)]) + sim2
        m = jnp.min(d, axis=1, keepdims=True)
        iota = lax.broadcasted_iota(jnp.int32, (RB, KB), 1) + off
        bidx = jnp.min(jnp.where(d == m, iota, K), axis=1, keepdims=True)
        better = m < rmin
        return jnp.where(better, m, rmin), jnp.where(better, bidx, ridx)

    rmin0 = jnp.full((RB, 1), jnp.inf, jnp.float32)
    ridx0 = jnp.zeros((RB, 1), jnp.int32)
    rmin, ridx = lax.fori_loop(0, NK, dist_step, (rmin0, ridx0))
    idx_ref[...] = ridx
    loss_ref[...] = rmin * scale


def _encode_indices(x, embedding, scale):
    N, D = x.shape
    K = embedding.shape[1]
    RB = 1024 if N % 1024 == 0 else N
    KB = 4096 if K % 4096 == 0 else K
    NR, NK = N // RB, K // KB
    esq, em2 = pl.pallas_call(
        _prep_body,
        out_shape=[
            jax.ShapeDtypeStruct((1, K), jnp.float32),
            jax.ShapeDtypeStruct((D, K), jnp.bfloat16),
        ],
    )(embedding)
    body = functools.partial(_argmin_body, RB=RB, KB=KB, NK=NK, K=K, scale=scale)
    idx, loss = pl.pallas_call(
        body,
        grid=(NR,),
        in_specs=[
            pl.BlockSpec((RB, D), lambda i: (i, 0)),
            pl.BlockSpec((1, K), lambda i: (0, 0)),
            pl.BlockSpec((D, K), lambda i: (0, 0)),
        ],
        out_specs=[
            pl.BlockSpec((RB, 1), lambda i: (i, 0)),
            pl.BlockSpec((RB, 1), lambda i: (i, 0)),
        ],
        out_shape=[
            jax.ShapeDtypeStruct((N, 1), jnp.int32),
            jax.ShapeDtypeStruct((N, 1), jnp.float32),
        ],
        compiler_params=pltpu.CompilerParams(
            dimension_semantics=("parallel",)),
    )(x, esq, em2)
    return idx.reshape(N), jnp.sum(loss)


def _sc_gather(e_t, idx, N, D):
    info = plsc.get_sparse_core_info()
    NC, NS = info.num_cores, info.num_subcores
    NW = NC * NS
    BPW = N // NW
    CH = min(BPW, 128)
    NCH = BPW // CH
    mesh = plsc.VectorSubcoreMesh(core_axis_name="c", subcore_axis_name="s")

    @functools.partial(
        pl.kernel,
        out_type=jax.ShapeDtypeStruct((N, D), jnp.float32),
        mesh=mesh,
        scratch_types=[
            [pltpu.VMEM((CH,), jnp.int32)] * NCH,
            [pltpu.VMEM((CH, D), jnp.float32)] * NCH,
            [pltpu.SemaphoreType.DMA] * NCH,
            [pltpu.SemaphoreType.DMA] * NCH,
        ],
    )
    def sc_kernel(et_hbm, idx_hbm, out_hbm, idx_vs, q_vs, gsems, wsems):
        wid = lax.axis_index("s") * NC + lax.axis_index("c")
        base = wid * BPW
        gathers = []
        for ch in range(NCH):
            cb = base + ch * CH
            pltpu.sync_copy(idx_hbm.at[pl.ds(cb, CH)], idx_vs[ch])
            gathers.append(pltpu.async_copy(
                et_hbm.at[idx_vs[ch]], q_vs[ch], gsems[ch]))
        writes = []
        for ch in range(NCH):
            cb = base + ch * CH
            gathers[ch].wait()
            writes.append(pltpu.async_copy(
                q_vs[ch], out_hbm.at[pl.ds(cb, CH)], wsems[ch]))
        for w in writes:
            w.wait()

    return sc_kernel(e_t, idx)


def kernel(inputs, embedding):
    orig_shape = inputs.shape
    x = inputs.reshape(-1, orig_shape[-1])
    N, D = x.shape
    e_t = embedding.T.astype(jnp.bfloat16).astype(jnp.float32)
    scale = (1.0 + BETA) / float(inputs.size)
    idx, loss = _encode_indices(x, embedding, scale)
    out = _sc_gather(e_t, idx, N, D)
    return out.reshape(orig_shape), loss


# speedup vs baseline: 1.4349x; 1.0423x over previous
import functools

import jax
import jax.numpy as jnp
from jax import lax
from jax.experimental import pallas as pl
from jax.experimental.pallas import tpu as pltpu
from jax.experimental.pallas import tpu_sc as plsc

BETA = 0.25


def _argmin_body(x_ref, e_ref, idx_ref, loss_ref, esq_ref, em2_ref,
                 *, RB, KB, NK, K, scale):
    i = pl.program_id(0)

    @pl.when(i == 0)
    def _():
        e = e_ref[...]
        esq_ref[...] = jnp.sum(e * e, axis=0, keepdims=True)
        em2_ref[...] = (e * -2.0).astype(jnp.bfloat16)
        loss_ref[...] = jnp.zeros_like(loss_ref)

    x = x_ref[...]
    xb = x.astype(jnp.bfloat16)
    xsq = jnp.sum(x * x, axis=1, keepdims=True)

    def dist_step(kb, carry):
        rmin, ridx = carry
        off = kb * KB
        sim2 = lax.dot_general(
            xb, em2_ref[:, pl.ds(off, KB)], (((1,), (0,)), ((), ())),
            preferred_element_type=jnp.float32)
        d = (xsq + esq_ref[:, pl.ds(off, KB)]) + sim2
        m = jnp.min(d, axis=1, keepdims=True)
        iota = lax.broadcasted_iota(jnp.int32, (RB, KB), 1)
        bidx = jnp.min(jnp.where(d == m, iota, K), axis=1, keepdims=True) + off
        better = m < rmin
        return jnp.where(better, m, rmin), jnp.where(better, bidx, ridx)

    rmin0 = jnp.full((RB, 1), jnp.inf, jnp.float32)
    ridx0 = jnp.zeros((RB, 1), jnp.int32)
    rmin, ridx = lax.fori_loop(0, NK, dist_step, (rmin0, ridx0))
    idx_ref[...] = ridx
    loss_ref[...] += jnp.sum(rmin, axis=0, keepdims=True) * scale


def _encode_indices(x, embedding, scale):
    N, D = x.shape
    K = embedding.shape[1]
    RB = 2048 if N % 2048 == 0 else N
    KB = 4096 if K % 4096 == 0 else K
    NR, NK = N // RB, K // KB
    body = functools.partial(_argmin_body, RB=RB, KB=KB, NK=NK, K=K, scale=scale)
    idx, loss = pl.pallas_call(
        body,
        grid=(NR,),
        in_specs=[
            pl.BlockSpec((RB, D), lambda i: (i, 0)),
            pl.BlockSpec((D, K), lambda i: (0, 0)),
        ],
        out_specs=[
            pl.BlockSpec((RB, 1), lambda i: (i, 0)),
            pl.BlockSpec((1, 1), lambda i: (0, 0)),
        ],
        out_shape=[
            jax.ShapeDtypeStruct((N, 1), jnp.int32),
            jax.ShapeDtypeStruct((1, 1), jnp.float32),
        ],
        scratch_shapes=[
            pltpu.VMEM((1, K), jnp.float32),
            pltpu.VMEM((D, K), jnp.bfloat16),
        ],
        compiler_params=pltpu.CompilerParams(
            dimension_semantics=("arbitrary",)),
    )(x, embedding)
    return idx.reshape(N), loss.reshape(())


def _sc_gather(e_t, idx, N, D):
    info = plsc.get_sparse_core_info()
    NC, NS = info.num_cores, info.num_subcores
    NW = NC * NS
    BPW = N // NW
    CH = min(BPW, 128)
    NCH = BPW // CH
    mesh = plsc.VectorSubcoreMesh(core_axis_name="c", subcore_axis_name="s")

    @functools.partial(
        pl.kernel,
        out_type=jax.ShapeDtypeStruct((N, D), jnp.float32),
        mesh=mesh,
        scratch_types=[
            [pltpu.VMEM((CH,), jnp.int32)] * NCH,
            [pltpu.VMEM((CH, D), jnp.float32)] * NCH,
            [pltpu.SemaphoreType.DMA] * NCH,
            [pltpu.SemaphoreType.DMA] * NCH,
        ],
    )
    def sc_kernel(et_hbm, idx_hbm, out_hbm, idx_vs, q_vs, gsems, wsems):
        wid = lax.axis_index("s") * NC + lax.axis_index("c")
        base = wid * BPW
        gathers = []
        for ch in range(NCH):
            cb = base + ch * CH
            pltpu.sync_copy(idx_hbm.at[pl.ds(cb, CH)], idx_vs[ch])
            gathers.append(pltpu.async_copy(
                et_hbm.at[idx_vs[ch]], q_vs[ch], gsems[ch]))
        writes = []
        for ch in range(NCH):
            cb = base + ch * CH
            gathers[ch].wait()
            writes.append(pltpu.async_copy(
                q_vs[ch], out_hbm.at[pl.ds(cb, CH)], wsems[ch]))
        for w in writes:
            w.wait()

    return sc_kernel(e_t, idx)


def kernel(inputs, embedding):
    orig_shape = inputs.shape
    x = inputs.reshape(-1, orig_shape[-1])
    N, D = x.shape
    e_t = embedding.T.astype(jnp.bfloat16).astype(jnp.float32)
    scale = (1.0 + BETA) / float(inputs.size)
    idx, loss = _encode_indices(x, embedding, scale)
    out = _sc_gather(e_t, idx, N, D)
    return out.reshape(orig_shape), loss
